# baseline (device time: 75820 ns/iter reference)
import jax
import jax.numpy as jnp
from jax import lax
from jax.experimental import pallas as pl
from jax.experimental.pallas import tpu as pltpu

N_DEV = 16
W_BUFS = 2


def kernel(x, w_mat):
    m_total, k_per = x.shape
    k_total, n_out = w_mat.shape
    m_per = m_total // N_DEV

    def body(x_ref, w_ref, out_ref, recv_buf, a_stage, w_buf,
             send_sems, recv_sems, w_sems):
        my_i = lax.axis_index("i")

        def start_w(t):
            dmas = []
            for p in range(2):
                j = (my_i - (2 * t + p)) % N_DEV
                dma = pltpu.make_async_copy(
                    w_ref.at[pl.ds(j * k_per, k_per), :],
                    w_buf.at[t % W_BUFS, pl.ds(p * k_per, k_per), :],
                    w_sems.at[t % W_BUFS],
                )
                dma.start()
                dmas.append(dma)
            return dmas

        n_pairs = N_DEV // 2
        w_dmas = [None] * n_pairs
        for t in range(W_BUFS):
            w_dmas[t] = start_w(t)

        barrier_sem = pltpu.get_barrier_semaphore()
        for dj in range(1, N_DEV):
            peer = (my_i + dj) % N_DEV
            pl.semaphore_signal(
                barrier_sem, inc=1,
                device_id=(peer,), device_id_type=pl.DeviceIdType.MESH,
            )
        pl.semaphore_wait(barrier_sem, N_DEV - 1)

        rdmas = [None] * N_DEV
        for dj in range(1, N_DEV):
            dst = (my_i + dj) % N_DEV
            rdma = pltpu.make_async_remote_copy(
                src_ref=x_ref.at[pl.ds(dst * m_per, m_per), :],
                dst_ref=recv_buf.at[dj],
                send_sem=send_sems.at[dj],
                recv_sem=recv_sems.at[dj],
                device_id=(dst,),
                device_id_type=pl.DeviceIdType.MESH,
            )
            rdma.start()
            rdmas[dj] = rdma

        for t in range(n_pairs):
            for dma in w_dmas[t]:
                dma.wait()
            for p in range(2):
                k = 2 * t + p
                if k == 0:
                    a_stage[t % 2, :, p * k_per:(p + 1) * k_per] = (
                        x_ref[pl.ds(my_i * m_per, m_per), :]
                    )
                else:
                    rdmas[k].wait_recv()
                    a_stage[t % 2, :, p * k_per:(p + 1) * k_per] = recv_buf[k]
            acc = jnp.dot(
                a_stage[t % 2], w_buf[t % W_BUFS],
                preferred_element_type=jnp.float32,
            )
            if t == 0:
                out_ref[...] = acc
            else:
                out_ref[...] += acc
            if t + W_BUFS < n_pairs:
                w_dmas[t + W_BUFS] = start_w(t + W_BUFS)

        y = out_ref[...]
        c = 0.7978845608028654
        out_ref[...] = 0.5 * y * (1.0 + jnp.tanh(c * (y + 0.044715 * y * y * y)))

        for dj in range(1, N_DEV):
            rdmas[dj].wait_send()

    return pl.pallas_call(
        body,
        out_shape=jax.ShapeDtypeStruct((m_per, n_out), jnp.float32),
        in_specs=[
            pl.BlockSpec(memory_space=pltpu.VMEM),
            pl.BlockSpec(memory_space=pl.ANY),
        ],
        out_specs=pl.BlockSpec(memory_space=pltpu.VMEM),
        scratch_shapes=[
            pltpu.VMEM((N_DEV, m_per, k_per), jnp.float32),
            pltpu.VMEM((2, m_per, 2 * k_per), jnp.float32),
            pltpu.VMEM((W_BUFS, 2 * k_per, n_out), jnp.float32),
            pltpu.SemaphoreType.DMA((N_DEV,)),
            pltpu.SemaphoreType.DMA((N_DEV,)),
            pltpu.SemaphoreType.DMA((W_BUFS,)),
        ],
        compiler_params=pltpu.CompilerParams(
            collective_id=0,
            vmem_limit_bytes=100 * 1024 * 1024,
        ),
    )(x, w_mat)


# device time: 68638 ns/iter; 1.1046x vs baseline; 1.1046x over previous
import jax
import jax.numpy as jnp
from jax import lax
from jax.experimental import pallas as pl
from jax.experimental.pallas import tpu as pltpu

N_DEV = 16
W_BUFS = 3


def kernel(x, w_mat):
    m_total, k_per = x.shape
    k_total, n_out = w_mat.shape
    m_per = m_total // N_DEV

    def body(x_ref, w_ref, out_ref, recv_buf, w_buf, send_sems, recv_sems, w_sems):
        my_i = lax.axis_index("i")

        def start_w(k):
            j = (my_i - k) % N_DEV
            dma = pltpu.make_async_copy(
                w_ref.at[pl.ds(j * k_per, k_per), :],
                w_buf.at[k % W_BUFS],
                w_sems.at[k % W_BUFS],
            )
            dma.start()
            return dma

        w_dmas = [None] * N_DEV
        for k in range(W_BUFS):
            w_dmas[k] = start_w(k)

        barrier_sem = pltpu.get_barrier_semaphore()
        for dj in range(1, N_DEV):
            peer = (my_i + dj) % N_DEV
            pl.semaphore_signal(
                barrier_sem, inc=1,
                device_id=(peer,), device_id_type=pl.DeviceIdType.MESH,
            )
        pl.semaphore_wait(barrier_sem, N_DEV - 1)

        rdmas = [None] * N_DEV
        for dj in range(1, N_DEV):
            dst = (my_i + dj) % N_DEV
            rdma = pltpu.make_async_remote_copy(
                src_ref=x_ref.at[pl.ds(dst * m_per, m_per), :],
                dst_ref=recv_buf.at[dj],
                send_sem=send_sems.at[dj],
                recv_sem=recv_sems.at[dj],
                device_id=(dst,),
                device_id_type=pl.DeviceIdType.MESH,
            )
            rdma.start()
            rdmas[dj] = rdma

        for k in range(N_DEV):
            w_dmas[k].wait()
            if k == 0:
                a = x_ref[pl.ds(my_i * m_per, m_per), :]
            else:
                rdmas[k].wait_recv()
                a = recv_buf[k]
            acc = jnp.dot(a, w_buf[k % W_BUFS], preferred_element_type=jnp.float32)
            if k == 0:
                out_ref[...] = acc
            else:
                out_ref[...] += acc
            if k + W_BUFS < N_DEV:
                w_dmas[k + W_BUFS] = start_w(k + W_BUFS)

        y = out_ref[...]
        c = 0.7978845608028654
        out_ref[...] = 0.5 * y * (1.0 + jnp.tanh(c * (y + 0.044715 * y * y * y)))

        for dj in range(1, N_DEV):
            rdmas[dj].wait_send()

    return pl.pallas_call(
        body,
        out_shape=jax.ShapeDtypeStruct((m_per, n_out), jnp.float32),
        in_specs=[
            pl.BlockSpec(memory_space=pltpu.VMEM),
            pl.BlockSpec(memory_space=pl.ANY),
        ],
        out_specs=pl.BlockSpec(memory_space=pltpu.VMEM),
        scratch_shapes=[
            pltpu.VMEM((N_DEV, m_per, k_per), jnp.float32),
            pltpu.VMEM((W_BUFS, k_per, n_out), jnp.float32),
            pltpu.SemaphoreType.DMA((N_DEV,)),
            pltpu.SemaphoreType.DMA((N_DEV,)),
            pltpu.SemaphoreType.DMA((W_BUFS,)),
        ],
        compiler_params=pltpu.CompilerParams(collective_id=0),
    )(x, w_mat)
